# baseline (speedup 1.0000x reference)
import functools

import jax
import jax.numpy as jnp
from jax import lax
from jax.experimental import pallas as pl
from jax.experimental.pallas import tpu as pltpu
from jax.experimental.pallas import tpu_sc as plsc

N = 10000
E = 320000
D = 128
G = 8

NC = 2
NS = 16
NW = NC * NS
CH = 128

N_CH = 80
QCH = 16
E_PER_W = N_CH * CH
E_PAD = NW * E_PER_W
N_ACC = ((N + 16 + 8 * NS - 1) // (8 * NS)) * (8 * NS)
ROWS_PER_SUB = N_ACC // NS

DW = 128

BN = 1000
NB = N // BN


_MESH = plsc.VectorSubcoreMesh(
    core_axis_name="c", subcore_axis_name="s", num_cores=NC, num_subcores=NS)


def _sc_degree_body(dst_hbm, ones_hbm, zeros_hbm, out_hbm, didx, ones_v, acc,
                    sem):
    c = lax.axis_index("c")
    s = lax.axis_index("s")
    wid = s * NC + c
    pltpu.sync_copy(zeros_hbm, acc.at[pl.ds(s * ROWS_PER_SUB, ROWS_PER_SUB)])
    pltpu.sync_copy(ones_hbm, ones_v)
    pltpu.sync_copy(dst_hbm.at[pl.ds(wid * N_CH, N_CH)], didx)
    plsc.subcore_barrier()

    for k in range(4):
        pltpu.async_copy(ones_v, acc.at[didx.at[k]], sem, add=True)

    def chunk(i, carry):
        pltpu.make_async_copy(ones_v, acc.at[didx.at[i]], sem).wait()
        pltpu.async_copy(ones_v, acc.at[didx.at[i + 4]], sem, add=True)
        return carry

    lax.fori_loop(0, N_CH - 4, chunk, 0)
    for k in range(4):
        pltpu.make_async_copy(ones_v, acc.at[didx.at[k]], sem).wait()
    plsc.subcore_barrier()
    pltpu.sync_copy(acc.at[pl.ds(s * ROWS_PER_SUB, ROWS_PER_SUB)],
                    out_hbm.at[c, pl.ds(s * ROWS_PER_SUB, ROWS_PER_SUB)])


def _sc_aggregate_body(y_hbm, src_hbm, dst_hbm, zeros_hbm, out_hbm,
                       sidx, didx, rows0, rows1, acc,
                       gsem0, gsem1, ssem0, ssem1):
    c = lax.axis_index("c")
    s = lax.axis_index("s")
    wid = s * NC + c
    pltpu.sync_copy(zeros_hbm, acc.at[pl.ds(s * ROWS_PER_SUB, ROWS_PER_SUB)])
    plsc.subcore_barrier()

    def quarter(q, carry):
        pltpu.sync_copy(src_hbm.at[pl.ds(wid * N_CH + q * QCH, QCH)], sidx)
        pltpu.sync_copy(dst_hbm.at[pl.ds(wid * N_CH + q * QCH, QCH)], didx)
        pltpu.async_copy(y_hbm.at[sidx.at[0]], rows0, gsem0)

        def pair(j, carry2):
            i = 2 * j
            pltpu.async_copy(y_hbm.at[sidx.at[i + 1]], rows1, gsem1)
            pltpu.make_async_copy(y_hbm.at[sidx.at[i]], rows0, gsem0).wait()
            pltpu.sync_copy(rows0, acc.at[didx.at[i]], add=True)

            @pl.when(j < QCH // 2 - 1)
            def _():
                pltpu.async_copy(y_hbm.at[sidx.at[i + 2]], rows0, gsem0)

            pltpu.make_async_copy(y_hbm.at[sidx.at[i + 1]], rows1, gsem1).wait()
            pltpu.sync_copy(rows1, acc.at[didx.at[i + 1]], add=True)
            return carry2

        lax.fori_loop(0, QCH // 2, pair, 0)
        return carry

    lax.fori_loop(0, N_CH // QCH, quarter, 0)
    plsc.subcore_barrier()
    pltpu.sync_copy(acc.at[pl.ds(s * ROWS_PER_SUB, ROWS_PER_SUB)],
                    out_hbm.at[c, pl.ds(s * ROWS_PER_SUB, ROWS_PER_SUB)])


def _make_sc_degree(interpret=False):
    return pl.kernel(
        _sc_degree_body,
        out_type=jax.ShapeDtypeStruct((NC, N_ACC, D), jnp.float32),
        mesh=_MESH,
        scratch_types=[
            pltpu.VMEM((N_CH, CH), jnp.int32),
            pltpu.VMEM((CH, DW), jnp.float32),
            pltpu.VMEM_SHARED((N_ACC, DW), jnp.float32),
            pltpu.SemaphoreType.DMA,
        ],
        interpret=interpret,
    )


def _make_sc_aggregate(interpret=False):
    return pl.kernel(
        _sc_aggregate_body,
        out_type=jax.ShapeDtypeStruct((NC, N_ACC, D), jnp.float32),
        mesh=_MESH,
        scratch_types=[
            pltpu.VMEM((QCH, CH), jnp.int32),
            pltpu.VMEM((QCH, CH), jnp.int32),
            pltpu.VMEM((CH, D), jnp.float32),
            pltpu.VMEM((CH, D), jnp.float32),
            pltpu.VMEM_SHARED((N_ACC, D), jnp.float32),
            pltpu.SemaphoreType.DMA,
            pltpu.SemaphoreType.DMA,
            pltpu.SemaphoreType.DMA,
            pltpu.SemaphoreType.DMA,
        ],
        interpret=interpret,
    )


_sc_degree = _make_sc_degree()
_sc_aggregate = _make_sc_aggregate()


def _tc_prep_body(cnt_ref, x_ref, w_ref, dinv_ref, y_ref):
    c = cnt_ref[0] + cnt_ref[1]
    deg = 1.0 + c[:, 0:1]
    dinv = lax.rsqrt(deg)
    xw = jnp.dot(x_ref[...], w_ref[...], preferred_element_type=jnp.float32)
    dinv_ref[...] = jnp.broadcast_to(dinv, (BN, 16))
    y_ref[...] = xw * jnp.broadcast_to(dinv, (BN, D))


_tc_prep = pl.pallas_call(
    _tc_prep_body,
    grid=(NB,),
    in_specs=[
        pl.BlockSpec((NC, BN, D), lambda i: (0, i, 0)),
        pl.BlockSpec((BN, D), lambda i: (i, 0)),
        pl.BlockSpec((D, D), lambda i: (0, 0)),
    ],
    out_specs=[
        pl.BlockSpec((BN, 16), lambda i: (i, 0)),
        pl.BlockSpec((BN, D), lambda i: (i, 0)),
    ],
    out_shape=[
        jax.ShapeDtypeStruct((N, 16), jnp.float32),
        jax.ShapeDtypeStruct((N, D), jnp.float32),
    ],
)


def _pool_accum(pool_ref, h, batch_blk):
    rows = [jnp.sum(jnp.where(batch_blk == g, h, 0.0), axis=0)
            for g in range(G)]
    contrib = jnp.stack(rows, axis=0)

    @pl.when(pl.program_id(0) == 0)
    def _():
        pool_ref[...] = jnp.zeros((G, D), jnp.float32)

    pool_ref[...] += contrib


def _tc_mid_body(parts_ref, y_ref, dinv_ref, b_ref, w_ref, batch_ref,
                 ynext_ref, pool_ref):
    dinvb = jnp.broadcast_to(dinv_ref[...][:, 0:1], (BN, D))
    ssum = parts_ref[0] + parts_ref[1]
    h = jnp.maximum(dinvb * (ssum + y_ref[...]) + b_ref[...], 0.0)
    ynext_ref[...] = jnp.dot(
        h, w_ref[...], preferred_element_type=jnp.float32) * dinvb
    _pool_accum(pool_ref, h, batch_ref[...])


_tc_mid = pl.pallas_call(
    _tc_mid_body,
    grid=(NB,),
    in_specs=[
        pl.BlockSpec((NC, BN, D), lambda i: (0, i, 0)),
        pl.BlockSpec((BN, D), lambda i: (i, 0)),
        pl.BlockSpec((BN, 16), lambda i: (i, 0)),
        pl.BlockSpec((1, D), lambda i: (0, 0)),
        pl.BlockSpec((D, D), lambda i: (0, 0)),
        pl.BlockSpec((BN, 1), lambda i: (i, 0)),
    ],
    out_specs=[
        pl.BlockSpec((BN, D), lambda i: (i, 0)),
        pl.BlockSpec((G, D), lambda i: (0, 0)),
    ],
    out_shape=[
        jax.ShapeDtypeStruct((N, D), jnp.float32),
        jax.ShapeDtypeStruct((G, D), jnp.float32),
    ],
)


def _tc_last_body(parts_ref, y_ref, dinv_ref, b_ref, batch_ref, pool_ref):
    dinvb = jnp.broadcast_to(dinv_ref[...][:, 0:1], (BN, D))
    ssum = parts_ref[0] + parts_ref[1]
    h = jnp.maximum(dinvb * (ssum + y_ref[...]) + b_ref[...], 0.0)
    _pool_accum(pool_ref, h, batch_ref[...])


_tc_last = pl.pallas_call(
    _tc_last_body,
    grid=(NB,),
    in_specs=[
        pl.BlockSpec((NC, BN, D), lambda i: (0, i, 0)),
        pl.BlockSpec((BN, D), lambda i: (i, 0)),
        pl.BlockSpec((BN, 16), lambda i: (i, 0)),
        pl.BlockSpec((1, D), lambda i: (0, 0)),
        pl.BlockSpec((BN, 1), lambda i: (i, 0)),
    ],
    out_specs=pl.BlockSpec((G, D), lambda i: (0, 0)),
    out_shape=jax.ShapeDtypeStruct((G, D), jnp.float32),
)


def kernel(x, edge_index, batch, W1, b1, W2, b2, W3, b3):
    src = edge_index[0].astype(jnp.int32)
    dst = edge_index[1].astype(jnp.int32)

    pad = E_PAD - E
    pad_ar = jnp.arange(pad, dtype=jnp.int32)
    src_p = jnp.concatenate([src, (pad_ar * 97) % N]).reshape(NW * N_CH, CH)
    dst_p = jnp.concatenate([dst, N + (pad_ar % 64)]).reshape(NW * N_CH, CH)

    zeros_sm = jnp.zeros((ROWS_PER_SUB, D), jnp.float32)
    ones_dw = jnp.ones((CH, DW), jnp.float32)
    batch2 = batch.astype(jnp.int32).reshape(N, 1)

    cnt = _sc_degree(dst_p, ones_dw, zeros_sm)
    dinvb, y1 = _tc_prep(cnt, x, W1)

    p1 = _sc_aggregate(y1, src_p, dst_p, zeros_sm)
    y2, pool1 = _tc_mid(p1, y1, dinvb, b1.reshape(1, D), W2, batch2)

    p2 = _sc_aggregate(y2, src_p, dst_p, zeros_sm)
    y3, pool2 = _tc_mid(p2, y2, dinvb, b2.reshape(1, D), W3, batch2)

    p3 = _sc_aggregate(y3, src_p, dst_p, zeros_sm)
    pool3 = _tc_last(p3, y3, dinvb, b3.reshape(1, D), batch2)

    return jnp.concatenate([pool1, pool2, pool3], axis=1)


# speedup vs baseline: 1.0630x; 1.0630x over previous
import functools

import jax
import jax.numpy as jnp
from jax import lax
from jax.experimental import pallas as pl
from jax.experimental.pallas import tpu as pltpu
from jax.experimental.pallas import tpu_sc as plsc

N = 10000
E = 320000
D = 128
G = 8

NC = 2
NS = 16
NW = NC * NS
CH = 128

N_CH = 80
QCH = 40
E_PER_W = N_CH * CH
E_PAD = NW * E_PER_W
N_ACC = ((N + 16 + 8 * NS - 1) // (8 * NS)) * (8 * NS)
ROWS_PER_SUB = N_ACC // NS

DW = 128

BN = 1000
NB = N // BN


_MESH = plsc.VectorSubcoreMesh(
    core_axis_name="c", subcore_axis_name="s", num_cores=NC, num_subcores=NS)


def _sc_degree_body(dst_hbm, ones_hbm, zeros_hbm, out_hbm, didx, ones_v, acc,
                    sem):
    c = lax.axis_index("c")
    s = lax.axis_index("s")
    wid = s * NC + c
    pltpu.sync_copy(zeros_hbm.at[pl.ds(s * ROWS_PER_SUB, ROWS_PER_SUB)],
                    acc.at[pl.ds(s * ROWS_PER_SUB, ROWS_PER_SUB)])
    pltpu.sync_copy(ones_hbm, ones_v)
    pltpu.sync_copy(dst_hbm.at[pl.ds(wid * N_CH, N_CH)], didx)
    plsc.subcore_barrier()

    def chunk(i, carry):
        pltpu.sync_copy(ones_v, acc.at[didx.at[i]], add=True)
        return carry

    lax.fori_loop(0, N_CH, chunk, 0)
    plsc.subcore_barrier()
    pltpu.sync_copy(acc.at[pl.ds(s * ROWS_PER_SUB, ROWS_PER_SUB)],
                    out_hbm.at[c, pl.ds(s * ROWS_PER_SUB, ROWS_PER_SUB)])


def _sc_aggregate_body(y_hbm, src_hbm, dst_hbm, zeros_hbm, out_hbm,
                       sidx, didx, rows0, rows1, acc,
                       gsem0, gsem1, ssem0, ssem1):
    c = lax.axis_index("c")
    s = lax.axis_index("s")
    wid = s * NC + c
    pltpu.sync_copy(zeros_hbm.at[pl.ds(s * ROWS_PER_SUB, ROWS_PER_SUB)],
                    acc.at[pl.ds(s * ROWS_PER_SUB, ROWS_PER_SUB)])
    plsc.subcore_barrier()

    def quarter(q, carry):
        pltpu.sync_copy(src_hbm.at[pl.ds(wid * N_CH + q * QCH, QCH)], sidx)
        pltpu.sync_copy(dst_hbm.at[pl.ds(wid * N_CH + q * QCH, QCH)], didx)
        pltpu.async_copy(y_hbm.at[sidx.at[0]], rows0, gsem0)

        def pair(j, carry2):
            i = 2 * j
            pltpu.async_copy(y_hbm.at[sidx.at[i + 1]], rows1, gsem1)
            pltpu.make_async_copy(y_hbm.at[sidx.at[i]], rows0, gsem0).wait()
            pltpu.sync_copy(rows0, acc.at[didx.at[i]], add=True)

            @pl.when(j < QCH // 2 - 1)
            def _():
                pltpu.async_copy(y_hbm.at[sidx.at[i + 2]], rows0, gsem0)

            pltpu.make_async_copy(y_hbm.at[sidx.at[i + 1]], rows1, gsem1).wait()
            pltpu.sync_copy(rows1, acc.at[didx.at[i + 1]], add=True)
            return carry2

        lax.fori_loop(0, QCH // 2, pair, 0)
        return carry

    lax.fori_loop(0, N_CH // QCH, quarter, 0)
    plsc.subcore_barrier()
    pltpu.sync_copy(acc.at[pl.ds(s * ROWS_PER_SUB, ROWS_PER_SUB)],
                    out_hbm.at[c, pl.ds(s * ROWS_PER_SUB, ROWS_PER_SUB)])


def _make_sc_degree(interpret=False):
    return pl.kernel(
        _sc_degree_body,
        out_type=jax.ShapeDtypeStruct((NC, N_ACC, D), jnp.float32),
        mesh=_MESH,
        scratch_types=[
            pltpu.VMEM((N_CH, CH), jnp.int32),
            pltpu.VMEM((CH, DW), jnp.float32),
            pltpu.VMEM_SHARED((N_ACC, DW), jnp.float32),
            pltpu.SemaphoreType.DMA,
        ],
        interpret=interpret,
    )


def _make_sc_aggregate(interpret=False):
    return pl.kernel(
        _sc_aggregate_body,
        out_type=jax.ShapeDtypeStruct((NC, N_ACC, D), jnp.float32),
        mesh=_MESH,
        scratch_types=[
            pltpu.VMEM((QCH, CH), jnp.int32),
            pltpu.VMEM((QCH, CH), jnp.int32),
            pltpu.VMEM((CH, D), jnp.float32),
            pltpu.VMEM((CH, D), jnp.float32),
            pltpu.VMEM_SHARED((N_ACC, D), jnp.float32),
            pltpu.SemaphoreType.DMA,
            pltpu.SemaphoreType.DMA,
            pltpu.SemaphoreType.DMA,
            pltpu.SemaphoreType.DMA,
        ],
        interpret=interpret,
    )


_sc_degree = _make_sc_degree()
_sc_aggregate = _make_sc_aggregate()


def _tc_prep_body(cnt_ref, x_ref, w_ref, dinv_ref, y_ref):
    c = cnt_ref[0] + cnt_ref[1]
    deg = 1.0 + c[:, 0:1]
    dinv = lax.rsqrt(deg)
    xw = jnp.dot(x_ref[...], w_ref[...], preferred_element_type=jnp.float32)
    dinvb = jnp.broadcast_to(dinv, (BN, D))
    dinv_ref[...] = dinvb
    y_ref[...] = xw * dinvb


_tc_prep = pl.pallas_call(
    _tc_prep_body,
    grid=(NB,),
    in_specs=[
        pl.BlockSpec((NC, BN, D), lambda i: (0, i, 0)),
        pl.BlockSpec((BN, D), lambda i: (i, 0)),
        pl.BlockSpec((D, D), lambda i: (0, 0)),
    ],
    out_specs=[
        pl.BlockSpec((BN, D), lambda i: (i, 0)),
        pl.BlockSpec((BN, D), lambda i: (i, 0)),
    ],
    out_shape=[
        jax.ShapeDtypeStruct((N, D), jnp.float32),
        jax.ShapeDtypeStruct((N, D), jnp.float32),
    ],
)


def _pool_accum(pool_ref, h, batch_blk):
    rows = [jnp.sum(jnp.where(batch_blk == g, h, 0.0), axis=0)
            for g in range(G)]
    contrib = jnp.stack(rows, axis=0)

    @pl.when(pl.program_id(0) == 0)
    def _():
        pool_ref[...] = jnp.zeros((G, D), jnp.float32)

    pool_ref[...] += contrib


def _tc_mid_body(parts_ref, y_ref, dinv_ref, b_ref, w_ref, batch_ref,
                 ynext_ref, pool_ref):
    dinvb = dinv_ref[...]
    ssum = parts_ref[0] + parts_ref[1]
    h = jnp.maximum(dinvb * (ssum + y_ref[...]) + b_ref[...], 0.0)
    ynext_ref[...] = jnp.dot(
        h, w_ref[...], preferred_element_type=jnp.float32) * dinvb
    _pool_accum(pool_ref, h, batch_ref[...])


_tc_mid = pl.pallas_call(
    _tc_mid_body,
    grid=(NB,),
    in_specs=[
        pl.BlockSpec((NC, BN, D), lambda i: (0, i, 0)),
        pl.BlockSpec((BN, D), lambda i: (i, 0)),
        pl.BlockSpec((BN, D), lambda i: (i, 0)),
        pl.BlockSpec((1, D), lambda i: (0, 0)),
        pl.BlockSpec((D, D), lambda i: (0, 0)),
        pl.BlockSpec((BN, 1), lambda i: (i, 0)),
    ],
    out_specs=[
        pl.BlockSpec((BN, D), lambda i: (i, 0)),
        pl.BlockSpec((G, D), lambda i: (0, 0)),
    ],
    out_shape=[
        jax.ShapeDtypeStruct((N, D), jnp.float32),
        jax.ShapeDtypeStruct((G, D), jnp.float32),
    ],
)


def _tc_last_body(parts_ref, y_ref, dinv_ref, b_ref, batch_ref, pool_ref):
    dinvb = dinv_ref[...]
    ssum = parts_ref[0] + parts_ref[1]
    h = jnp.maximum(dinvb * (ssum + y_ref[...]) + b_ref[...], 0.0)
    _pool_accum(pool_ref, h, batch_ref[...])


_tc_last = pl.pallas_call(
    _tc_last_body,
    grid=(NB,),
    in_specs=[
        pl.BlockSpec((NC, BN, D), lambda i: (0, i, 0)),
        pl.BlockSpec((BN, D), lambda i: (i, 0)),
        pl.BlockSpec((BN, D), lambda i: (i, 0)),
        pl.BlockSpec((1, D), lambda i: (0, 0)),
        pl.BlockSpec((BN, 1), lambda i: (i, 0)),
    ],
    out_specs=pl.BlockSpec((G, D), lambda i: (0, 0)),
    out_shape=jax.ShapeDtypeStruct((G, D), jnp.float32),
)


def kernel(x, edge_index, batch, W1, b1, W2, b2, W3, b3):
    src = edge_index[0].astype(jnp.int32)
    dst = edge_index[1].astype(jnp.int32)

    pad = E_PAD - E
    pad_ar = jnp.arange(pad, dtype=jnp.int32)
    src_p = jnp.concatenate([src, pad_ar]).reshape(NW * N_CH, CH)
    dst_p = jnp.concatenate(
        [dst, N + (pad_ar & 63)]).reshape(NW * N_CH, CH)

    zeros_sm = jnp.zeros((N_ACC, D), jnp.float32)
    ones_dw = jnp.ones((CH, DW), jnp.float32)
    batch2 = batch.astype(jnp.int32).reshape(N, 1)

    cnt = _sc_degree(dst_p, ones_dw, zeros_sm)
    dinvb, y1 = _tc_prep(cnt, x, W1)

    p1 = _sc_aggregate(y1, src_p, dst_p, zeros_sm)
    y2, pool1 = _tc_mid(p1, y1, dinvb, b1.reshape(1, D), W2, batch2)

    p2 = _sc_aggregate(y2, src_p, dst_p, zeros_sm)
    y3, pool2 = _tc_mid(p2, y2, dinvb, b2.reshape(1, D), W3, batch2)

    p3 = _sc_aggregate(y3, src_p, dst_p, zeros_sm)
    pool3 = _tc_last(p3, y3, dinvb, b3.reshape(1, D), batch2)

    return jnp.concatenate([pool1, pool2, pool3], axis=1)


# speedup vs baseline: 1.0648x; 1.0017x over previous
import functools

import jax
import jax.numpy as jnp
from jax import lax
from jax.experimental import pallas as pl
from jax.experimental.pallas import tpu as pltpu
from jax.experimental.pallas import tpu_sc as plsc

N = 10000
E = 320000
D = 128
G = 8

NC = 2
NS = 16
NW = NC * NS
CH = 128

N_CH = 80
QCH = 40
E_PER_W = N_CH * CH
E_PAD = NW * E_PER_W
N_ACC = ((N + 16 + 8 * NS - 1) // (8 * NS)) * (8 * NS)
ROWS_PER_SUB = N_ACC // NS

DW = 128

BN = 1000
NB = N // BN


_MESH = plsc.VectorSubcoreMesh(
    core_axis_name="c", subcore_axis_name="s", num_cores=NC, num_subcores=NS)


def _sc_degree_body(dst_hbm, ones_hbm, zeros_hbm, out_hbm, didx, ones_v, acc,
                    sem):
    c = lax.axis_index("c")
    s = lax.axis_index("s")
    wid = s * NC + c
    pltpu.sync_copy(zeros_hbm.at[pl.ds(s * ROWS_PER_SUB, ROWS_PER_SUB)],
                    acc.at[pl.ds(s * ROWS_PER_SUB, ROWS_PER_SUB)])
    pltpu.sync_copy(ones_hbm, ones_v)
    pltpu.sync_copy(dst_hbm.at[pl.ds(wid * N_CH, N_CH)], didx)
    plsc.subcore_barrier()

    def chunk(i, carry):
        pltpu.sync_copy(ones_v, acc.at[didx.at[i]], add=True)
        return carry

    lax.fori_loop(0, N_CH, chunk, 0)
    plsc.subcore_barrier()
    pltpu.sync_copy(acc.at[pl.ds(s * ROWS_PER_SUB, ROWS_PER_SUB)],
                    out_hbm.at[c, pl.ds(s * ROWS_PER_SUB, ROWS_PER_SUB)])


def _sc_aggregate_body(y_hbm, src_hbm, dst_hbm, zeros_hbm, out_hbm,
                       sidx, didx, rows0, rows1, acc,
                       gsem0, gsem1, ssem0, ssem1):
    c = lax.axis_index("c")
    s = lax.axis_index("s")
    wid = s * NC + c
    pltpu.sync_copy(zeros_hbm.at[pl.ds(s * ROWS_PER_SUB, ROWS_PER_SUB)],
                    acc.at[pl.ds(s * ROWS_PER_SUB, ROWS_PER_SUB)])
    plsc.subcore_barrier()

    def quarter(q, carry):
        pltpu.sync_copy(src_hbm.at[pl.ds(wid * N_CH + q * QCH, QCH)], sidx)
        pltpu.sync_copy(dst_hbm.at[pl.ds(wid * N_CH + q * QCH, QCH)], didx)
        pltpu.async_copy(y_hbm.at[sidx.at[0]], rows0, gsem0)

        def pair(j, carry2):
            i = 2 * j
            pltpu.async_copy(y_hbm.at[sidx.at[i + 1]], rows1, gsem1)
            pltpu.make_async_copy(y_hbm.at[sidx.at[i]], rows0, gsem0).wait()
            pltpu.sync_copy(rows0, acc.at[didx.at[i]], add=True)

            @pl.when(j < QCH // 2 - 1)
            def _():
                pltpu.async_copy(y_hbm.at[sidx.at[i + 2]], rows0, gsem0)

            pltpu.make_async_copy(y_hbm.at[sidx.at[i + 1]], rows1, gsem1).wait()
            pltpu.sync_copy(rows1, acc.at[didx.at[i + 1]], add=True)
            return carry2

        lax.fori_loop(0, QCH // 2, pair, 0)
        return carry

    lax.fori_loop(0, N_CH // QCH, quarter, 0)
    plsc.subcore_barrier()
    pltpu.sync_copy(acc.at[pl.ds(s * ROWS_PER_SUB, ROWS_PER_SUB)],
                    out_hbm.at[c, pl.ds(s * ROWS_PER_SUB, ROWS_PER_SUB)])


def _make_sc_degree(interpret=False):
    return pl.kernel(
        _sc_degree_body,
        out_type=jax.ShapeDtypeStruct((NC, N_ACC, D), jnp.float32),
        mesh=_MESH,
        scratch_types=[
            pltpu.VMEM((N_CH, CH), jnp.int32),
            pltpu.VMEM((CH, DW), jnp.float32),
            pltpu.VMEM_SHARED((N_ACC, DW), jnp.float32),
            pltpu.SemaphoreType.DMA,
        ],
        compiler_params=pltpu.CompilerParams(use_tc_tiling_on_sc=True),
        interpret=interpret,
    )


def _make_sc_aggregate(interpret=False):
    return pl.kernel(
        _sc_aggregate_body,
        out_type=jax.ShapeDtypeStruct((NC, N_ACC, D), jnp.float32),
        mesh=_MESH,
        scratch_types=[
            pltpu.VMEM((QCH, CH), jnp.int32),
            pltpu.VMEM((QCH, CH), jnp.int32),
            pltpu.VMEM((CH, D), jnp.float32),
            pltpu.VMEM((CH, D), jnp.float32),
            pltpu.VMEM_SHARED((N_ACC, D), jnp.float32),
            pltpu.SemaphoreType.DMA,
            pltpu.SemaphoreType.DMA,
            pltpu.SemaphoreType.DMA,
            pltpu.SemaphoreType.DMA,
        ],
        compiler_params=pltpu.CompilerParams(use_tc_tiling_on_sc=True),
        interpret=interpret,
    )


_sc_degree = _make_sc_degree()
_sc_aggregate = _make_sc_aggregate()


def _tc_prep_body(cnt_ref, x_ref, w_ref, dinv_ref, y_ref):
    c = cnt_ref[0] + cnt_ref[1]
    deg = 1.0 + c[:, 0:1]
    dinv = lax.rsqrt(deg)
    xw = jnp.dot(x_ref[...], w_ref[...], preferred_element_type=jnp.float32)
    dinvb = jnp.broadcast_to(dinv, (BN, D))
    dinv_ref[...] = dinvb
    y_ref[...] = xw * dinvb


_tc_prep = pl.pallas_call(
    _tc_prep_body,
    grid=(NB,),
    in_specs=[
        pl.BlockSpec((NC, BN, D), lambda i: (0, i, 0)),
        pl.BlockSpec((BN, D), lambda i: (i, 0)),
        pl.BlockSpec((D, D), lambda i: (0, 0)),
    ],
    out_specs=[
        pl.BlockSpec((BN, D), lambda i: (i, 0)),
        pl.BlockSpec((BN, D), lambda i: (i, 0)),
    ],
    out_shape=[
        jax.ShapeDtypeStruct((N, D), jnp.float32),
        jax.ShapeDtypeStruct((N, D), jnp.float32),
    ],
)


def _pool_accum(pool_ref, h, batch_blk):
    rows = [jnp.sum(jnp.where(batch_blk == g, h, 0.0), axis=0)
            for g in range(G)]
    contrib = jnp.stack(rows, axis=0)

    @pl.when(pl.program_id(0) == 0)
    def _():
        pool_ref[...] = jnp.zeros((G, D), jnp.float32)

    pool_ref[...] += contrib


def _tc_mid_body(parts_ref, y_ref, dinv_ref, b_ref, w_ref, batch_ref,
                 ynext_ref, pool_ref):
    dinvb = dinv_ref[...]
    ssum = parts_ref[0] + parts_ref[1]
    h = jnp.maximum(dinvb * (ssum + y_ref[...]) + b_ref[...], 0.0)
    ynext_ref[...] = jnp.dot(
        h, w_ref[...], preferred_element_type=jnp.float32) * dinvb
    _pool_accum(pool_ref, h, batch_ref[...])


_tc_mid = pl.pallas_call(
    _tc_mid_body,
    grid=(NB,),
    in_specs=[
        pl.BlockSpec((NC, BN, D), lambda i: (0, i, 0)),
        pl.BlockSpec((BN, D), lambda i: (i, 0)),
        pl.BlockSpec((BN, D), lambda i: (i, 0)),
        pl.BlockSpec((1, D), lambda i: (0, 0)),
        pl.BlockSpec((D, D), lambda i: (0, 0)),
        pl.BlockSpec((BN, 1), lambda i: (i, 0)),
    ],
    out_specs=[
        pl.BlockSpec((BN, D), lambda i: (i, 0)),
        pl.BlockSpec((G, D), lambda i: (0, 0)),
    ],
    out_shape=[
        jax.ShapeDtypeStruct((N, D), jnp.float32),
        jax.ShapeDtypeStruct((G, D), jnp.float32),
    ],
)


def _tc_last_body(parts_ref, y_ref, dinv_ref, b_ref, batch_ref, pool_ref):
    dinvb = dinv_ref[...]
    ssum = parts_ref[0] + parts_ref[1]
    h = jnp.maximum(dinvb * (ssum + y_ref[...]) + b_ref[...], 0.0)
    _pool_accum(pool_ref, h, batch_ref[...])


_tc_last = pl.pallas_call(
    _tc_last_body,
    grid=(NB,),
    in_specs=[
        pl.BlockSpec((NC, BN, D), lambda i: (0, i, 0)),
        pl.BlockSpec((BN, D), lambda i: (i, 0)),
        pl.BlockSpec((BN, D), lambda i: (i, 0)),
        pl.BlockSpec((1, D), lambda i: (0, 0)),
        pl.BlockSpec((BN, 1), lambda i: (i, 0)),
    ],
    out_specs=pl.BlockSpec((G, D), lambda i: (0, 0)),
    out_shape=jax.ShapeDtypeStruct((G, D), jnp.float32),
)


def kernel(x, edge_index, batch, W1, b1, W2, b2, W3, b3):
    src = edge_index[0].astype(jnp.int32)
    dst = edge_index[1].astype(jnp.int32)

    pad = E_PAD - E
    pad_ar = jnp.arange(pad, dtype=jnp.int32)
    src_p = jnp.concatenate([src, pad_ar]).reshape(NW * N_CH, CH)
    dst_p = jnp.concatenate(
        [dst, N + (pad_ar & 63)]).reshape(NW * N_CH, CH)

    zeros_sm = jnp.zeros((N_ACC, D), jnp.float32)
    ones_dw = jnp.ones((CH, DW), jnp.float32)
    batch2 = batch.astype(jnp.int32).reshape(N, 1)

    cnt = _sc_degree(dst_p, ones_dw, zeros_sm)
    dinvb, y1 = _tc_prep(cnt, x, W1)

    p1 = _sc_aggregate(y1, src_p, dst_p, zeros_sm)
    y2, pool1 = _tc_mid(p1, y1, dinvb, b1.reshape(1, D), W2, batch2)

    p2 = _sc_aggregate(y2, src_p, dst_p, zeros_sm)
    y3, pool2 = _tc_mid(p2, y2, dinvb, b2.reshape(1, D), W3, batch2)

    p3 = _sc_aggregate(y3, src_p, dst_p, zeros_sm)
    pool3 = _tc_last(p3, y3, dinvb, b3.reshape(1, D), batch2)

    return jnp.concatenate([pool1, pool2, pool3], axis=1)


# speedup vs baseline: 1.0649x; 1.0001x over previous
import jax
import jax.numpy as jnp
from jax import lax
from jax.experimental import pallas as pl
from jax.experimental.pallas import tpu as pltpu
from jax.experimental.pallas import tpu_sc as plsc

N = 10000
E = 320000
D = 128
G = 8

NC = 2
NS = 16
NW = NC * NS
CH = 128

N_CH = 80
QCH = 40
E_PER_W = N_CH * CH
E_PAD = NW * E_PER_W
N_ACC = ((N + 16 + 8 * NS - 1) // (8 * NS)) * (8 * NS)
ROWS_PER_SUB = N_ACC // NS

DW = 128

BN = 1000
NB = N // BN


_MESH = plsc.VectorSubcoreMesh(
    core_axis_name="c", subcore_axis_name="s", num_cores=NC, num_subcores=NS)


def _sc_degree_body(dst_hbm, ones_hbm, zeros_hbm, out_hbm, didx, ones_v, acc,
                    sem):
    c = lax.axis_index("c")
    s = lax.axis_index("s")
    wid = s * NC + c
    pltpu.sync_copy(zeros_hbm.at[pl.ds(s * ROWS_PER_SUB, ROWS_PER_SUB)],
                    acc.at[pl.ds(s * ROWS_PER_SUB, ROWS_PER_SUB)])
    pltpu.sync_copy(ones_hbm, ones_v)
    pltpu.sync_copy(dst_hbm.at[pl.ds(wid * N_CH, N_CH)], didx)
    plsc.subcore_barrier()

    def chunk(i, carry):
        pltpu.sync_copy(ones_v, acc.at[didx.at[i]], add=True)
        return carry

    lax.fori_loop(0, N_CH, chunk, 0)
    plsc.subcore_barrier()
    pltpu.sync_copy(acc.at[pl.ds(s * ROWS_PER_SUB, ROWS_PER_SUB)],
                    out_hbm.at[c, pl.ds(s * ROWS_PER_SUB, ROWS_PER_SUB)])


def _sc_aggregate_body(y_hbm, src_hbm, dst_hbm, zeros_hbm, out_hbm,
                       sidx, didx, rows0, rows1, acc, gsem0, gsem1):
    c = lax.axis_index("c")
    s = lax.axis_index("s")
    wid = s * NC + c
    pltpu.sync_copy(zeros_hbm.at[pl.ds(s * ROWS_PER_SUB, ROWS_PER_SUB)],
                    acc.at[pl.ds(s * ROWS_PER_SUB, ROWS_PER_SUB)])
    plsc.subcore_barrier()

    def quarter(q, carry):
        pltpu.sync_copy(src_hbm.at[pl.ds(wid * N_CH + q * QCH, QCH)], sidx)
        pltpu.sync_copy(dst_hbm.at[pl.ds(wid * N_CH + q * QCH, QCH)], didx)
        pltpu.async_copy(y_hbm.at[sidx.at[0]], rows0, gsem0)

        def pair(j, carry2):
            i = 2 * j
            pltpu.async_copy(y_hbm.at[sidx.at[i + 1]], rows1, gsem1)
            pltpu.make_async_copy(y_hbm.at[sidx.at[i]], rows0, gsem0).wait()
            pltpu.sync_copy(rows0, acc.at[didx.at[i]], add=True)

            @pl.when(j < QCH // 2 - 1)
            def _():
                pltpu.async_copy(y_hbm.at[sidx.at[i + 2]], rows0, gsem0)

            pltpu.make_async_copy(y_hbm.at[sidx.at[i + 1]], rows1, gsem1).wait()
            pltpu.sync_copy(rows1, acc.at[didx.at[i + 1]], add=True)
            return carry2

        lax.fori_loop(0, QCH // 2, pair, 0)
        return carry

    lax.fori_loop(0, N_CH // QCH, quarter, 0)
    plsc.subcore_barrier()
    pltpu.sync_copy(acc.at[pl.ds(s * ROWS_PER_SUB, ROWS_PER_SUB)],
                    out_hbm.at[c, pl.ds(s * ROWS_PER_SUB, ROWS_PER_SUB)])


def _make_sc_degree(interpret=False):
    return pl.kernel(
        _sc_degree_body,
        out_type=jax.ShapeDtypeStruct((NC, N_ACC, D), jnp.float32),
        mesh=_MESH,
        scratch_types=[
            pltpu.VMEM((N_CH, CH), jnp.int32),
            pltpu.VMEM((CH, DW), jnp.float32),
            pltpu.VMEM_SHARED((N_ACC, DW), jnp.float32),
            pltpu.SemaphoreType.DMA,
        ],
        compiler_params=pltpu.CompilerParams(use_tc_tiling_on_sc=True),
        interpret=interpret,
    )


def _make_sc_aggregate(interpret=False):
    return pl.kernel(
        _sc_aggregate_body,
        out_type=jax.ShapeDtypeStruct((NC, N_ACC, D), jnp.float32),
        mesh=_MESH,
        scratch_types=[
            pltpu.VMEM((QCH, CH), jnp.int32),
            pltpu.VMEM((QCH, CH), jnp.int32),
            pltpu.VMEM((CH, D), jnp.float32),
            pltpu.VMEM((CH, D), jnp.float32),
            pltpu.VMEM_SHARED((N_ACC, D), jnp.float32),
            pltpu.SemaphoreType.DMA,
            pltpu.SemaphoreType.DMA,
        ],
        compiler_params=pltpu.CompilerParams(use_tc_tiling_on_sc=True),
        interpret=interpret,
    )


_sc_degree = _make_sc_degree()
_sc_aggregate = _make_sc_aggregate()


def _tc_prep_body(cnt_ref, x_ref, w_ref, dinv_ref, y_ref):
    c = cnt_ref[0] + cnt_ref[1]
    deg = 1.0 + c[:, 0:1]
    dinv = lax.rsqrt(deg)
    xw = jnp.dot(x_ref[...], w_ref[...], preferred_element_type=jnp.float32)
    dinvb = jnp.broadcast_to(dinv, (BN, D))
    dinv_ref[...] = dinvb
    y_ref[...] = xw * dinvb


_tc_prep = pl.pallas_call(
    _tc_prep_body,
    grid=(NB,),
    in_specs=[
        pl.BlockSpec((NC, BN, D), lambda i: (0, i, 0)),
        pl.BlockSpec((BN, D), lambda i: (i, 0)),
        pl.BlockSpec((D, D), lambda i: (0, 0)),
    ],
    out_specs=[
        pl.BlockSpec((BN, D), lambda i: (i, 0)),
        pl.BlockSpec((BN, D), lambda i: (i, 0)),
    ],
    out_shape=[
        jax.ShapeDtypeStruct((N, D), jnp.float32),
        jax.ShapeDtypeStruct((N, D), jnp.float32),
    ],
)


def _pool_accum(pool_ref, h, batch_blk):
    rows = [jnp.sum(jnp.where(batch_blk == g, h, 0.0), axis=0)
            for g in range(G)]
    contrib = jnp.stack(rows, axis=0)

    @pl.when(pl.program_id(0) == 0)
    def _():
        pool_ref[...] = jnp.zeros((G, D), jnp.float32)

    pool_ref[...] += contrib


def _tc_mid_body(parts_ref, y_ref, dinv_ref, b_ref, w_ref, batch_ref,
                 ynext_ref, pool_ref):
    dinvb = dinv_ref[...]
    ssum = parts_ref[0] + parts_ref[1]
    h = jnp.maximum(dinvb * (ssum + y_ref[...]) + b_ref[...], 0.0)
    ynext_ref[...] = jnp.dot(
        h, w_ref[...], preferred_element_type=jnp.float32) * dinvb
    _pool_accum(pool_ref, h, batch_ref[...])


_tc_mid = pl.pallas_call(
    _tc_mid_body,
    grid=(NB,),
    in_specs=[
        pl.BlockSpec((NC, BN, D), lambda i: (0, i, 0)),
        pl.BlockSpec((BN, D), lambda i: (i, 0)),
        pl.BlockSpec((BN, D), lambda i: (i, 0)),
        pl.BlockSpec((1, D), lambda i: (0, 0)),
        pl.BlockSpec((D, D), lambda i: (0, 0)),
        pl.BlockSpec((BN, 1), lambda i: (i, 0)),
    ],
    out_specs=[
        pl.BlockSpec((BN, D), lambda i: (i, 0)),
        pl.BlockSpec((G, D), lambda i: (0, 0)),
    ],
    out_shape=[
        jax.ShapeDtypeStruct((N, D), jnp.float32),
        jax.ShapeDtypeStruct((G, D), jnp.float32),
    ],
)


def _tc_last_body(parts_ref, y_ref, dinv_ref, b_ref, batch_ref, pool_ref):
    dinvb = dinv_ref[...]
    ssum = parts_ref[0] + parts_ref[1]
    h = jnp.maximum(dinvb * (ssum + y_ref[...]) + b_ref[...], 0.0)
    _pool_accum(pool_ref, h, batch_ref[...])


_tc_last = pl.pallas_call(
    _tc_last_body,
    grid=(NB,),
    in_specs=[
        pl.BlockSpec((NC, BN, D), lambda i: (0, i, 0)),
        pl.BlockSpec((BN, D), lambda i: (i, 0)),
        pl.BlockSpec((BN, D), lambda i: (i, 0)),
        pl.BlockSpec((1, D), lambda i: (0, 0)),
        pl.BlockSpec((BN, 1), lambda i: (i, 0)),
    ],
    out_specs=pl.BlockSpec((G, D), lambda i: (0, 0)),
    out_shape=jax.ShapeDtypeStruct((G, D), jnp.float32),
)


def kernel(x, edge_index, batch, W1, b1, W2, b2, W3, b3):
    src = edge_index[0].astype(jnp.int32)
    dst = edge_index[1].astype(jnp.int32)

    pad = E_PAD - E
    pad_ar = jnp.arange(pad, dtype=jnp.int32)
    src_p = jnp.concatenate([src, pad_ar]).reshape(NW * N_CH, CH)
    dst_p = jnp.concatenate(
        [dst, N + (pad_ar & 63)]).reshape(NW * N_CH, CH)

    zeros_sm = jnp.zeros((N_ACC, D), jnp.float32)
    ones_dw = jnp.ones((CH, DW), jnp.float32)
    batch2 = batch.astype(jnp.int32).reshape(N, 1)

    cnt = _sc_degree(dst_p, ones_dw, zeros_sm)
    dinvb, y1 = _tc_prep(cnt, x, W1)

    p1 = _sc_aggregate(y1, src_p, dst_p, zeros_sm)
    y2, pool1 = _tc_mid(p1, y1, dinvb, b1.reshape(1, D), W2, batch2)

    p2 = _sc_aggregate(y2, src_p, dst_p, zeros_sm)
    y3, pool2 = _tc_mid(p2, y2, dinvb, b2.reshape(1, D), W3, batch2)

    p3 = _sc_aggregate(y3, src_p, dst_p, zeros_sm)
    pool3 = _tc_last(p3, y3, dinvb, b3.reshape(1, D), batch2)

    return jnp.concatenate([pool1, pool2, pool3], axis=1)
